# baseline (speedup 1.0000x reference)
import jax
import jax.numpy as jnp
from jax import lax
from jax.experimental import pallas as pl
from jax.experimental.pallas import tpu as pltpu
from jax.experimental.pallas import tpu_sc as plsc

N_NODES = 10000
D = 128
NC, NS = 2, 16
NW = NC * NS
CHUNK = 128
NCK = 80
NBUF_DEG = 4
N_PAD = 10240
BR = 1000

_mesh = plsc.VectorSubcoreMesh(core_axis_name="c", subcore_axis_name="s")


def _deg_body(dst_hbm, zeros_hbm, ones_hbm, out_hbm,
              didx_v, ones_v, deg_sh, *sems):
    c = lax.axis_index("c")
    s = lax.axis_index("s")
    w = s * NC + c
    rpt = N_PAD // NS
    pltpu.sync_copy(zeros_hbm.at[pl.ds(s * rpt, rpt)],
                    deg_sh.at[pl.ds(s * rpt, rpt)])
    pltpu.sync_copy(ones_hbm, ones_v)
    pltpu.sync_copy(dst_hbm.at[w], didx_v)
    plsc.subcore_barrier()

    def issue(g, k):
        pltpu.async_copy(ones_v, deg_sh.at[didx_v.at[g]], sems[k], add=True)

    def wait(g, k):
        pltpu.make_async_copy(ones_v, deg_sh.at[didx_v.at[g]],
                              sems[k]).wait()

    for k in range(NBUF_DEG):
        issue(k, k)

    def body(t, carry):
        g = NBUF_DEG + t * NBUF_DEG
        for k in range(NBUF_DEG):
            wait(g + k - NBUF_DEG, k)
            issue(g + k, k)
        return carry

    lax.fori_loop(0, NCK // NBUF_DEG - 1, body, 0)
    for k in range(NBUF_DEG):
        wait(NCK - NBUF_DEG + k, k)
    plsc.subcore_barrier()
    pltpu.sync_copy(deg_sh.at[pl.ds(s * rpt, rpt)],
                    out_hbm.at[c, pl.ds(s * rpt, rpt)])


def _msg_body(src_hbm, dst_hbm, hs_hbm, zeros_hbm, out_hbm,
              sidx_v, didx_v, rows_v, acc_sh, sem0, sem1, ssem0, ssem1):
    gsems = (sem0, sem1)
    ssems = (ssem0, ssem1)
    c = lax.axis_index("c")
    s = lax.axis_index("s")
    w = s * NC + c
    rpt = N_PAD // NS
    pltpu.sync_copy(zeros_hbm.at[pl.ds(s * rpt, rpt)],
                    acc_sh.at[pl.ds(s * rpt, rpt)])
    pltpu.sync_copy(src_hbm.at[w], sidx_v)
    plsc.subcore_barrier()

    def stage(g, k):
        pltpu.sync_copy(dst_hbm.at[w, g], didx_v.at[k])
        pltpu.async_copy(hs_hbm.at[sidx_v.at[g]], rows_v.at[k], gsems[k])

    def scatter(k):
        pltpu.make_async_copy(hs_hbm.at[sidx_v.at[0]], rows_v.at[k],
                              gsems[k]).wait()
        pltpu.async_copy(rows_v.at[k], acc_sh.at[didx_v.at[k]], ssems[k],
                         add=True)

    def swait(k):
        pltpu.make_async_copy(rows_v.at[k], acc_sh.at[didx_v.at[k]],
                              ssems[k]).wait()

    stage(0, 0)
    stage(1, 1)

    def body(t, carry):
        g = t * 2
        scatter(0)
        scatter(1)

        @pl.when(g + 2 < NCK)
        def _():
            swait(0)
            stage(g + 2, 0)

        @pl.when(g + 3 < NCK)
        def _():
            swait(1)
            stage(g + 3, 1)

        return carry

    lax.fori_loop(0, NCK // 2, body, 0)
    swait(0)
    swait(1)
    plsc.subcore_barrier()
    pltpu.sync_copy(acc_sh.at[pl.ds(s * rpt, rpt)],
                    out_hbm.at[c, pl.ds(s * rpt, rpt)])


def _hs_body(x_ref, w_ref, dcnt_ref, hs_ref):
    h = jnp.dot(x_ref[...], w_ref[...], preferred_element_type=jnp.float32)
    cnt = jnp.sum(dcnt_ref[...], axis=(0, 2))
    dinv = lax.rsqrt(cnt + 1.0)
    hs_ref[...] = h * dinv[:, None]


def _fin_body(p0_ref, p1_ref, hs_ref, dcnt_ref, b_ref, o_ref):
    acc = p0_ref[0] + p1_ref[0] + hs_ref[...]
    cnt = jnp.sum(dcnt_ref[...], axis=(0, 2))
    dinv = lax.rsqrt(cnt + 1.0)
    o = acc * dinv[:, None] + b_ref[...]
    m = jnp.max(o, axis=1, keepdims=True)
    ex = jnp.exp(o - m)
    lse = jnp.log(jnp.sum(ex, axis=1, keepdims=True))
    o_ref[...] = o - m - lse


def kernel(x, edge_index, W, b):
    E = edge_index.shape[1]
    e_pad = NW * CHUNK * NCK
    pad_e = e_pad - E
    pad_dst = N_NODES + jnp.arange(pad_e, dtype=jnp.int32) % (N_PAD - N_NODES)
    pad_src = jnp.arange(pad_e, dtype=jnp.int32) % N_NODES
    src = jnp.concatenate([edge_index[0].astype(jnp.int32), pad_src])
    dst = jnp.concatenate([edge_index[1].astype(jnp.int32), pad_dst])
    src3 = src.reshape(NW, NCK, CHUNK)
    dst3 = dst.reshape(NW, NCK, CHUNK)

    zeros1 = jnp.zeros((N_PAD,), jnp.float32)
    ones1 = jnp.ones((CHUNK,), jnp.float32)
    zerosD = jnp.zeros((N_PAD, D), jnp.float32)

    deg_kern = pl.kernel(
        _deg_body,
        out_type=jax.ShapeDtypeStruct((NC, N_PAD), jnp.float32),
        mesh=_mesh,
        scratch_types=[
            pltpu.VMEM((NCK, CHUNK), jnp.int32),
            pltpu.VMEM((CHUNK,), jnp.float32),
            pltpu.VMEM_SHARED((N_PAD,), jnp.float32),
        ] + [pltpu.SemaphoreType.DMA] * NBUF_DEG,
    )
    dcnt = deg_kern(dst3, zeros1, ones1).reshape(NC, N_PAD, 1)

    hs = pl.pallas_call(
        _hs_body,
        grid=(N_NODES // BR,),
        in_specs=[
            pl.BlockSpec((BR, D), lambda i: (i, 0)),
            pl.BlockSpec((D, D), lambda i: (0, 0)),
            pl.BlockSpec((NC, BR, 1), lambda i: (0, i, 0)),
        ],
        out_specs=pl.BlockSpec((BR, D), lambda i: (i, 0)),
        out_shape=jax.ShapeDtypeStruct((N_NODES, D), jnp.float32),
    )(x, W, dcnt)

    msg_kern = pl.kernel(
        _msg_body,
        out_type=jax.ShapeDtypeStruct((NC, N_PAD, D), jnp.float32),
        mesh=_mesh,
        scratch_types=[
            pltpu.VMEM((NCK, CHUNK), jnp.int32),
            pltpu.VMEM((2, CHUNK), jnp.int32),
            pltpu.VMEM((2, CHUNK, D), jnp.float32),
            pltpu.VMEM_SHARED((N_PAD, D), jnp.float32),
            pltpu.SemaphoreType.DMA,
            pltpu.SemaphoreType.DMA,
            pltpu.SemaphoreType.DMA,
            pltpu.SemaphoreType.DMA,
        ],
    )
    parts = msg_kern(src3, dst3, hs, zerosD)

    b2 = b.reshape(1, D)
    out = pl.pallas_call(
        _fin_body,
        grid=(N_NODES // BR,),
        in_specs=[
            pl.BlockSpec((1, BR, D), lambda i: (0, i, 0)),
            pl.BlockSpec((1, BR, D), lambda i: (1, i, 0)),
            pl.BlockSpec((BR, D), lambda i: (i, 0)),
            pl.BlockSpec((NC, BR, 1), lambda i: (0, i, 0)),
            pl.BlockSpec((1, D), lambda i: (0, 0)),
        ],
        out_specs=pl.BlockSpec((BR, D), lambda i: (i, 0)),
        out_shape=jax.ShapeDtypeStruct((N_NODES, D), jnp.float32),
    )(parts, parts, hs, dcnt, b2)

    return out


# speedup vs baseline: 1.1019x; 1.1019x over previous
import jax
import jax.numpy as jnp
from jax import lax
from jax.experimental import pallas as pl
from jax.experimental.pallas import tpu as pltpu
from jax.experimental.pallas import tpu_sc as plsc

N_NODES = 10000
D = 128
NC, NS = 2, 16
NW = NC * NS
CHUNK = 128
NCK = 80
NBUF_DEG = 4
N_PAD = 10240
BR = 1000

_mesh = plsc.VectorSubcoreMesh(core_axis_name="c", subcore_axis_name="s")


def _fill(ref, n, value):
    v = jnp.full((16,), value, jnp.float32)

    def body(i, carry):
        ref[pl.ds(i * 16, 16)] = v
        return carry

    lax.fori_loop(0, n // 16, body, 0)


def _deg_body(dst_hbm, out_hbm, didx_v, ones_v, zck_v, deg_sh, *sems):
    c = lax.axis_index("c")
    s = lax.axis_index("s")
    w = s * NC + c
    rpt = N_PAD // NS
    _fill(ones_v, CHUNK, 1.0)
    _fill(zck_v, rpt, 0.0)
    pltpu.sync_copy(dst_hbm.at[w], didx_v)
    pltpu.sync_copy(zck_v, deg_sh.at[pl.ds(s * rpt, rpt)])
    plsc.subcore_barrier()

    def issue(g, k):
        pltpu.async_copy(ones_v, deg_sh.at[didx_v.at[g]], sems[k], add=True)

    def wait(g, k):
        pltpu.make_async_copy(ones_v, deg_sh.at[didx_v.at[g]],
                              sems[k]).wait()

    for k in range(NBUF_DEG):
        issue(k, k)

    def body(t, carry):
        g = NBUF_DEG + t * NBUF_DEG
        for k in range(NBUF_DEG):
            wait(g + k - NBUF_DEG, k)
            issue(g + k, k)
        return carry

    lax.fori_loop(0, NCK // NBUF_DEG - 1, body, 0)
    for k in range(NBUF_DEG):
        wait(NCK - NBUF_DEG + k, k)
    plsc.subcore_barrier()
    pltpu.sync_copy(deg_sh.at[pl.ds(s * rpt, rpt)],
                    out_hbm.at[c, pl.ds(s * rpt, rpt)])


def _msg_body(src_hbm, dst_hbm, hs_hbm, out_hbm,
              sidx_v, didx_v, rows_v, acc_sh, sem0, sem1):
    gsems = (sem0, sem1)
    c = lax.axis_index("c")
    s = lax.axis_index("s")
    w = s * NC + c
    rpt = N_PAD // NS

    zero16 = jnp.zeros((16,), jnp.float32)

    def zbody(i, carry):
        rows_v[0, i >> 3, pl.ds((i & 7) * 16, 16)] = zero16
        return carry

    lax.fori_loop(0, CHUNK * (D // 16), zbody, 0)
    for j in range(rpt // CHUNK):
        pltpu.sync_copy(rows_v.at[0],
                        acc_sh.at[pl.ds(s * rpt + j * CHUNK, CHUNK)])
    pltpu.sync_copy(src_hbm.at[w], sidx_v)
    plsc.subcore_barrier()

    def stage(g, k):
        pltpu.sync_copy(dst_hbm.at[w, g], didx_v.at[k])
        pltpu.async_copy(hs_hbm.at[sidx_v.at[g]], rows_v.at[k], gsems[k])

    def drain(g, k):
        pltpu.make_async_copy(hs_hbm.at[sidx_v.at[g]], rows_v.at[k],
                              gsems[k]).wait()
        pltpu.sync_copy(rows_v.at[k], acc_sh.at[didx_v.at[k]], add=True)

    stage(0, 0)
    stage(1, 1)

    def body(t, carry):
        g = t * 2
        drain(g, 0)

        @pl.when(g + 2 < NCK)
        def _():
            stage(g + 2, 0)

        drain(g + 1, 1)

        @pl.when(g + 3 < NCK)
        def _():
            stage(g + 3, 1)

        return carry

    lax.fori_loop(0, NCK // 2, body, 0)
    plsc.subcore_barrier()
    pltpu.sync_copy(acc_sh.at[pl.ds(s * rpt, rpt)],
                    out_hbm.at[c, pl.ds(s * rpt, rpt)])


def _hs_body(x_ref, w_ref, dcnt_ref, hs_ref):
    h = jnp.dot(x_ref[...], w_ref[...], preferred_element_type=jnp.float32)
    cnt = jnp.sum(dcnt_ref[...], axis=(0, 2))
    dinv = lax.rsqrt(cnt + 1.0)
    hs_ref[...] = h * dinv[:, None]


def _fin_body(p0_ref, p1_ref, hs_ref, dcnt_ref, b_ref, o_ref):
    acc = p0_ref[0] + p1_ref[0] + hs_ref[...]
    cnt = jnp.sum(dcnt_ref[...], axis=(0, 2))
    dinv = lax.rsqrt(cnt + 1.0)
    o = acc * dinv[:, None] + b_ref[...]
    m = jnp.max(o, axis=1, keepdims=True)
    ex = jnp.exp(o - m)
    lse = jnp.log(jnp.sum(ex, axis=1, keepdims=True))
    o_ref[...] = o - m - lse


def kernel(x, edge_index, W, b):
    E = edge_index.shape[1]
    e_pad = NW * CHUNK * NCK
    pad_e = e_pad - E
    pad_dst = N_NODES + jnp.arange(pad_e, dtype=jnp.int32) % (N_PAD - N_NODES)
    pad_src = jnp.arange(pad_e, dtype=jnp.int32) % N_NODES
    src = jnp.concatenate([edge_index[0].astype(jnp.int32), pad_src])
    dst = jnp.concatenate([edge_index[1].astype(jnp.int32), pad_dst])
    src3 = src.reshape(NW, NCK, CHUNK)
    dst3 = dst.reshape(NW, NCK, CHUNK)

    deg_kern = pl.kernel(
        _deg_body,
        out_type=jax.ShapeDtypeStruct((NC, N_PAD), jnp.float32),
        mesh=_mesh,
        scratch_types=[
            pltpu.VMEM((NCK, CHUNK), jnp.int32),
            pltpu.VMEM((CHUNK,), jnp.float32),
            pltpu.VMEM((N_PAD // NS,), jnp.float32),
            pltpu.VMEM_SHARED((N_PAD,), jnp.float32),
        ] + [pltpu.SemaphoreType.DMA] * NBUF_DEG,
    )
    dcnt = deg_kern(dst3).reshape(NC, N_PAD, 1)

    hs = pl.pallas_call(
        _hs_body,
        grid=(N_NODES // BR,),
        in_specs=[
            pl.BlockSpec((BR, D), lambda i: (i, 0)),
            pl.BlockSpec((D, D), lambda i: (0, 0)),
            pl.BlockSpec((NC, BR, 1), lambda i: (0, i, 0)),
        ],
        out_specs=pl.BlockSpec((BR, D), lambda i: (i, 0)),
        out_shape=jax.ShapeDtypeStruct((N_NODES, D), jnp.float32),
    )(x, W, dcnt)

    msg_kern = pl.kernel(
        _msg_body,
        out_type=jax.ShapeDtypeStruct((NC, N_PAD, D), jnp.float32),
        mesh=_mesh,
        scratch_types=[
            pltpu.VMEM((NCK, CHUNK), jnp.int32),
            pltpu.VMEM((2, CHUNK), jnp.int32),
            pltpu.VMEM((2, CHUNK, D), jnp.float32),
            pltpu.VMEM_SHARED((N_PAD, D), jnp.float32),
            pltpu.SemaphoreType.DMA,
            pltpu.SemaphoreType.DMA,
        ],
    )
    parts = msg_kern(src3, dst3, hs)

    b2 = b.reshape(1, D)
    out = pl.pallas_call(
        _fin_body,
        grid=(N_NODES // BR,),
        in_specs=[
            pl.BlockSpec((1, BR, D), lambda i: (0, i, 0)),
            pl.BlockSpec((1, BR, D), lambda i: (1, i, 0)),
            pl.BlockSpec((BR, D), lambda i: (i, 0)),
            pl.BlockSpec((NC, BR, 1), lambda i: (0, i, 0)),
            pl.BlockSpec((1, D), lambda i: (0, 0)),
        ],
        out_specs=pl.BlockSpec((BR, D), lambda i: (i, 0)),
        out_shape=jax.ShapeDtypeStruct((N_NODES, D), jnp.float32),
    )(parts, parts, hs, dcnt, b2)

    return out
